# baseline (speedup 1.0000x reference)
import functools

import jax
import jax.numpy as jnp
from jax import lax
from jax.experimental import pallas as pl
from jax.experimental.pallas import tpu as pltpu
from jax.experimental.pallas import tpu_sc as plsc

N = 10000
E = 320000
D = 128
B = 1024
NPAD = 10240
NC = 2
NS = 16
NW = NC * NS
CH = 64
NCHUNKS = E // CH
FULL_ROUNDS = NCHUNKS // NW
EXTRA = NCHUNKS - FULL_ROUNDS * NW
N_PER_T = NPAD // NS
GCH = 128
NBLK = 1024
EBLK = 4000

_mesh = plsc.VectorSubcoreMesh(core_axis_name="c", subcore_axis_name="s")


@functools.partial(
    pl.kernel,
    out_type=(
        jax.ShapeDtypeStruct((NC, NPAD, D), jnp.float32),
        jax.ShapeDtypeStruct((NC, NS, 1, NPAD), jnp.float32),
    ),
    mesh=_mesh,
    scratch_types=[
        pltpu.VMEM((CH,), jnp.int32),
        pltpu.VMEM((CH,), jnp.int32),
        pltpu.VMEM((CH,), jnp.float32),
        pltpu.VMEM((CH, D), jnp.float32),
        pltpu.VMEM((CH, D), jnp.float32),
        pltpu.VMEM((CH,), jnp.int32),
        pltpu.VMEM((CH,), jnp.int32),
        pltpu.VMEM((CH,), jnp.int32),
        pltpu.VMEM((CH,), jnp.float32),
        pltpu.VMEM((CH, D), jnp.float32),
        pltpu.VMEM((CH, D), jnp.float32),
        pltpu.VMEM((CH,), jnp.int32),
        pltpu.VMEM((NPAD,), jnp.float32),
        pltpu.VMEM_SHARED((NPAD, D), jnp.float32),
        pltpu.SemaphoreType.DMA,
        pltpu.SemaphoreType.DMA,
        pltpu.SemaphoreType.DMA,
    ],
    compiler_params=pltpu.CompilerParams(needs_layout_passes=False),
)
def _sc_scatter(table, s_idx, d_idx, ew, out, out_cnt,
                is0, id0, ew0, rg0, rs0, ds0,
                is1, id1, ew1, rg1, rs1, ds1,
                cnt_v, acc_sh, gsem0, gsem1, ssem):
    c = lax.axis_index("c")
    s = lax.axis_index("s")
    wid = c * NS + s
    zeros16 = jnp.zeros((16,), jnp.float32)
    bufs = ((is0, id0, ew0, rg0, rs0, ds0, gsem0),
            (is1, id1, ew1, rg1, rs1, ds1, gsem1))

    def zcnt(i, carry):
        cnt_v[pl.ds(i * 16, 16)] = zeros16
        return carry

    lax.fori_loop(0, NPAD // 16, zcnt, 0)

    def zrow(i, carry):
        for j in range(D // 16):
            rs0[i, pl.ds(j * 16, 16)] = zeros16
        return carry

    lax.fori_loop(0, CH, zrow, 0)
    for k in range(N_PER_T // CH):
        pltpu.sync_copy(rs0, acc_sh.at[pl.ds(s * N_PER_T + k * CH, CH)])
    plsc.subcore_barrier()

    def load_chunk(r, isv, idv, ewv):
        off = (r * NW + wid) * CH
        pltpu.sync_copy(s_idx.at[pl.ds(off, CH)], isv)
        pltpu.sync_copy(d_idx.at[pl.ds(off, CH)], idv)
        pltpu.sync_copy(ew.at[pl.ds(off, CH)], ewv)

    def scale(idv, ewv, rg, rs, ds):
        def grp(g, carry):
            ew16 = ewv[pl.ds(g * 16, 16)]
            d16 = idv[pl.ds(g * 16, 16)]
            ds[pl.ds(g * 16, 16)] = d16
            plsc.addupdate_scatter(cnt_v, [d16], ew16)
            for j16 in range(16):
                i = g * 16 + j16
                w = lax.gather(
                    ew16, jnp.full((16, 1), j16, jnp.int32),
                    lax.GatherDimensionNumbers(
                        offset_dims=(), collapsed_slice_dims=(0,),
                        start_index_map=(0,)),
                    (1,), mode=lax.GatherScatterMode.PROMISE_IN_BOUNDS)
                for j in range(D // 16):
                    rs[i, pl.ds(j * 16, 16)] = rg[i, pl.ds(j * 16, 16)] * w
            return carry

        lax.fori_loop(0, CH // 16, grp, 0)

    def wait_rows(sem, buf):
        pltpu.make_async_copy(table.at[pl.ds(0, CH)], buf, sem).wait()

    load_chunk(0, is0, id0, ew0)
    pltpu.async_copy(table.at[is0], rg0, gsem0)

    def body(k, carry):
        for b in range(2):
            isv, idv, ewv, rg, rs, ds, gsem = bufs[b]
            isn, idn, ewn, rgn, _, _, gsemn = bufs[1 - b]
            r = 2 * k + b
            wait_rows(gsem, rg)
            if b == 0:
                load_chunk(r + 1, isn, idn, ewn)
                pltpu.async_copy(table.at[isn], rgn, gsemn)
            else:
                @pl.when(k < FULL_ROUNDS // 2 - 1)
                def _():
                    load_chunk(r + 1, isn, idn, ewn)
                    pltpu.async_copy(table.at[isn], rgn, gsemn)
            scale(idv, ewv, rg, rs, ds)
            if b == 0:
                @pl.when(k >= 1)
                def _():
                    wait_rows(ssem, rs1)
            else:
                wait_rows(ssem, rs0)
            pltpu.async_copy(rs, acc_sh.at[ds], ssem, add=True)
        return carry

    lax.fori_loop(0, FULL_ROUNDS // 2, body, 0)
    wait_rows(ssem, rs1)

    @pl.when(wid < EXTRA)
    def _():
        load_chunk(FULL_ROUNDS, is0, id0, ew0)
        pltpu.async_copy(table.at[is0], rg0, gsem0).wait()
        scale(id0, ew0, rg0, rs0, ds0)
        pltpu.sync_copy(rs0, acc_sh.at[ds0], add=True)

    plsc.subcore_barrier()

    pltpu.sync_copy(acc_sh.at[pl.ds(s * N_PER_T, N_PER_T)],
                    out.at[c, pl.ds(s * N_PER_T, N_PER_T)])
    pltpu.sync_copy(cnt_v, out_cnt.at[c, s, 0])


@functools.partial(
    pl.kernel,
    out_type=jax.ShapeDtypeStruct((2 * B, D), jnp.float32),
    mesh=_mesh,
    scratch_types=[
        pltpu.VMEM((GCH,), jnp.int32),
        pltpu.VMEM((GCH, D), jnp.float32),
        pltpu.SemaphoreType.DMA,
    ],
)
def _sc_gather(table, idx, out, idx_v, rows_v, sem):
    c = lax.axis_index("c")
    s = lax.axis_index("s")
    wid = c * NS + s

    @pl.when(wid < 2 * B // GCH)
    def _():
        base = wid * GCH
        pltpu.sync_copy(idx.at[pl.ds(base, GCH)], idx_v)
        pltpu.async_copy(table.at[idx_v], rows_v, sem).wait()
        pltpu.sync_copy(rows_v, out.at[pl.ds(base, GCH)])


def _proj_body(x_ref, w_ref, b_ref, out_ref):
    out_ref[...] = lax.dot_general(
        x_ref[...], w_ref[...], (((1,), (1,)), ((), ())),
        preferred_element_type=jnp.float32) + b_ref[...]


def _ew_body(ea_ref, q_ref, p0_ref, p1_ref, b0_ref, b1_ref, out_ref):
    wq = jnp.concatenate([p0_ref[...] * q_ref[...], p1_ref[...] * q_ref[...]],
                         axis=0)
    z = lax.dot_general(ea_ref[...], wq, (((1,), (1,)), ((), ())),
                        preferred_element_type=jnp.float32)
    b = jnp.concatenate([b0_ref[...], b1_ref[...]], axis=1)
    out_ref[...] = jax.nn.sigmoid(z + b)


def _gru_body(a0_ref, a1_ref, cnt_ref, h_ref, mw_ref, mb_ref, wih_ref,
              bih_ref, whh_ref, bhh_ref, out_ref):
    a = a0_ref[0] + a1_ref[0]
    mb32 = jnp.broadcast_to(mb_ref[...], (NW, D))
    bias = lax.dot_general(cnt_ref[...], mb32, (((0,), (0,)), ((), ())),
                           preferred_element_type=jnp.float32)
    aggr = lax.dot_general(a, mw_ref[...], (((1,), (1,)), ((), ())),
                           preferred_element_type=jnp.float32) + bias
    h = h_ref[...]
    gi = lax.dot_general(aggr, wih_ref[...], (((1,), (1,)), ((), ())),
                         preferred_element_type=jnp.float32) + bih_ref[...]
    gh = lax.dot_general(h, whh_ref[...], (((1,), (1,)), ((), ())),
                         preferred_element_type=jnp.float32) + bhh_ref[...]
    r = jax.nn.sigmoid(gi[:, :D] + gh[:, :D])
    z = jax.nn.sigmoid(gi[:, D:2 * D] + gh[:, D:2 * D])
    n = jnp.tanh(gi[:, 2 * D:] + r * gh[:, 2 * D:])
    out_ref[...] = (1.0 - z) * n + z * h


def _score_body(hsd_ref, lab_ref, q_ref, w1_ref, b1_ref, w2_ref, b2_ref,
                out_ref):
    hs = hsd_ref[:B, :]
    hd = hsd_ref[B:, :]
    qb = jnp.broadcast_to(q_ref[...], (B, D))
    feat = jnp.concatenate([hs, hd, hs * hd, qb], axis=1)
    hid = jax.nn.relu(
        lax.dot_general(feat, w1_ref[...], (((1,), (1,)), ((), ())),
                        preferred_element_type=jnp.float32) + b1_ref[...])
    logits = lax.dot_general(w2_ref[...], hid, (((1,), (1,)), ((), ())),
                             preferred_element_type=jnp.float32) + b2_ref[...]
    y = lab_ref[...].astype(jnp.float32)
    l = (jnp.maximum(logits, 0.0) - logits * y
         + jnp.log1p(jnp.exp(-jnp.abs(logits))))
    out_ref[...] = jnp.mean(l)[None, None]


def _full(shape):
    return pl.BlockSpec(shape, lambda i: tuple(0 for _ in shape))


_proj_call = pl.pallas_call(
    _proj_body,
    grid=(NPAD // NBLK,),
    in_specs=[
        pl.BlockSpec((NBLK, D), lambda i: (i, 0)),
        _full((D, D)),
        _full((1, D)),
    ],
    out_specs=pl.BlockSpec((NBLK, D), lambda i: (i, 0)),
    out_shape=jax.ShapeDtypeStruct((NPAD, D), jnp.float32),
)

_ew_call = pl.pallas_call(
    _ew_body,
    grid=(E // EBLK,),
    in_specs=[
        pl.BlockSpec((EBLK, D), lambda i: (i, 0)),
        _full((1, D)),
        _full((1, D)),
        _full((1, D)),
        _full((1, 1)),
        _full((1, 1)),
    ],
    out_specs=pl.BlockSpec((EBLK, 2), lambda i: (i, 0)),
    out_shape=jax.ShapeDtypeStruct((E, 2), jnp.float32),
)

_gru_call = pl.pallas_call(
    _gru_body,
    grid=(NPAD // NBLK,),
    in_specs=[
        pl.BlockSpec((1, NBLK, D), lambda i: (0, i, 0)),
        pl.BlockSpec((1, NBLK, D), lambda i: (1, i, 0)),
        pl.BlockSpec((NW, NBLK), lambda i: (0, i)),
        pl.BlockSpec((NBLK, D), lambda i: (i, 0)),
        _full((D, D)),
        _full((1, D)),
        _full((3 * D, D)),
        _full((1, 3 * D)),
        _full((3 * D, D)),
        _full((1, 3 * D)),
    ],
    out_specs=pl.BlockSpec((NBLK, D), lambda i: (i, 0)),
    out_shape=jax.ShapeDtypeStruct((NPAD, D), jnp.float32),
)

_score_call = pl.pallas_call(
    _score_body,
    grid=(1,),
    in_specs=[
        pl.BlockSpec((2 * B, D), lambda i: (0, 0)),
        pl.BlockSpec((1, B), lambda i: (0, 0)),
        _full((1, D)),
        _full((D, 4 * D)),
        _full((1, D)),
        _full((1, D)),
        _full((1, 1)),
    ],
    out_specs=pl.BlockSpec((1, 1), lambda i: (0, 0)),
    out_shape=jax.ShapeDtypeStruct((1, 1), jnp.float32),
)


def kernel(x, edge_index, edge_attr, q_emb, src, dst, labels, proj_W, proj_b,
           l0_msg_W, l0_msg_b, l0_phi_W, l0_phi_b, l0_gru_Wih, l0_gru_bih,
           l0_gru_Whh, l0_gru_bhh, l1_msg_W, l1_msg_b, l1_phi_W, l1_phi_b,
           l1_gru_Wih, l1_gru_bih, l1_gru_Whh, l1_gru_bhh, edge_W1, edge_b1,
           edge_W2, edge_b2):
    s_idx = edge_index[0]
    d_idx = edge_index[1]
    xp = jnp.pad(x, ((0, NPAD - N), (0, 0)))

    h0 = _proj_call(xp, proj_W, proj_b.reshape(1, D))
    ew = _ew_call(edge_attr, q_emb, l0_phi_W, l1_phi_W,
                  l0_phi_b.reshape(1, 1), l1_phi_b.reshape(1, 1))

    acc0, cnt0 = _sc_scatter(h0, s_idx, d_idx, ew[:, 0])
    h1 = _gru_call(acc0, acc0, cnt0.reshape(NW, NPAD), h0,
                   l0_msg_W, l0_msg_b.reshape(1, D),
                   l0_gru_Wih, l0_gru_bih.reshape(1, 3 * D),
                   l0_gru_Whh, l0_gru_bhh.reshape(1, 3 * D))

    acc1, cnt1 = _sc_scatter(h1, s_idx, d_idx, ew[:, 1])
    h2 = _gru_call(acc1, acc1, cnt1.reshape(NW, NPAD), h1,
                   l1_msg_W, l1_msg_b.reshape(1, D),
                   l1_gru_Wih, l1_gru_bih.reshape(1, 3 * D),
                   l1_gru_Whh, l1_gru_bhh.reshape(1, 3 * D))

    hsd = _sc_gather(h2, jnp.concatenate([src, dst]))
    loss = _score_call(hsd, labels.reshape(1, B), q_emb, edge_W1,
                       edge_b1.reshape(1, D), edge_W2, edge_b2.reshape(1, 1))
    return loss[0, 0]


# speedup vs baseline: 1.3132x; 1.3132x over previous
import functools

import jax
import jax.numpy as jnp
from jax import lax
from jax.experimental import pallas as pl
from jax.experimental.pallas import tpu as pltpu
from jax.experimental.pallas import tpu_sc as plsc

N = 10000
E = 320000
D = 128
B = 1024
NPAD = 10240
NC = 2
NS = 16
NW = NC * NS
CH = 128
NCHUNKS = E // CH
FULL_ROUNDS = NCHUNKS // NW
EXTRA = NCHUNKS - FULL_ROUNDS * NW
N_PER_T = NPAD // NS
GCH = 128
NBLK = 1024
EBLK = 4000

_mesh = plsc.VectorSubcoreMesh(core_axis_name="c", subcore_axis_name="s")


@functools.partial(
    pl.kernel,
    out_type=(
        jax.ShapeDtypeStruct((NC, NPAD, D), jnp.float32),
        jax.ShapeDtypeStruct((NC, NS, 1, NPAD), jnp.float32),
    ),
    mesh=_mesh,
    scratch_types=[
        pltpu.VMEM((CH,), jnp.int32),
        pltpu.VMEM((CH,), jnp.int32),
        pltpu.VMEM((CH,), jnp.float32),
        pltpu.VMEM((CH, D), jnp.float32),
        pltpu.VMEM((CH,), jnp.int32),
        pltpu.VMEM((CH,), jnp.int32),
        pltpu.VMEM((CH,), jnp.int32),
        pltpu.VMEM((CH,), jnp.float32),
        pltpu.VMEM((CH, D), jnp.float32),
        pltpu.VMEM((CH,), jnp.int32),
        pltpu.VMEM((NPAD,), jnp.float32),
        pltpu.VMEM_SHARED((NPAD, D), jnp.float32),
        pltpu.SemaphoreType.DMA,
        pltpu.SemaphoreType.DMA,
        pltpu.SemaphoreType.DMA,
    ],
    compiler_params=pltpu.CompilerParams(needs_layout_passes=False),
)
def _sc_scatter(table, s_idx, d_idx, ew, out, out_cnt,
                is0, id0, ew0, rg0, ds0,
                is1, id1, ew1, rg1, ds1,
                cnt_v, acc_sh, gsem0, gsem1, ssem):
    c = lax.axis_index("c")
    s = lax.axis_index("s")
    wid = c * NS + s
    zeros16 = jnp.zeros((16,), jnp.float32)
    bufs = ((is0, id0, ew0, rg0, ds0, gsem0),
            (is1, id1, ew1, rg1, ds1, gsem1))

    def zcnt(i, carry):
        cnt_v[pl.ds(i * 16, 16)] = zeros16
        return carry

    lax.fori_loop(0, NPAD // 16, zcnt, 0)

    def zrow(i, carry):
        for j in range(D // 16):
            rg0[i, pl.ds(j * 16, 16)] = zeros16
        return carry

    lax.fori_loop(0, CH, zrow, 0)
    for k in range(N_PER_T // CH):
        pltpu.sync_copy(rg0, acc_sh.at[pl.ds(s * N_PER_T + k * CH, CH)])
    plsc.subcore_barrier()

    def load_chunk(r, isv, idv, ewv):
        off = (r * NW + wid) * CH
        pltpu.sync_copy(s_idx.at[pl.ds(off, CH)], isv)
        pltpu.sync_copy(d_idx.at[pl.ds(off, CH)], idv)
        pltpu.sync_copy(ew.at[pl.ds(off, CH)], ewv)

    def scale(idv, ewv, rg, ds):
        def grp(g, carry):
            ew16 = ewv[pl.ds(g * 16, 16)]
            d16 = idv[pl.ds(g * 16, 16)]
            ds[pl.ds(g * 16, 16)] = d16
            plsc.addupdate_scatter(cnt_v, [d16], ew16)
            for j16 in range(16):
                i = g * 16 + j16
                w = lax.gather(
                    ew16, jnp.full((16, 1), j16, jnp.int32),
                    lax.GatherDimensionNumbers(
                        offset_dims=(), collapsed_slice_dims=(0,),
                        start_index_map=(0,)),
                    (1,), mode=lax.GatherScatterMode.PROMISE_IN_BOUNDS)
                for j in range(D // 16):
                    rg[i, pl.ds(j * 16, 16)] = rg[i, pl.ds(j * 16, 16)] * w
            return carry

        lax.fori_loop(0, CH // 16, grp, 0)

    def wait_rows(sem, buf):
        pltpu.make_async_copy(table.at[pl.ds(0, CH)], buf, sem).wait()

    load_chunk(0, is0, id0, ew0)
    pltpu.async_copy(table.at[is0], rg0, gsem0)

    def body(k, carry):
        for b in range(2):
            isv, idv, ewv, rg, ds, gsem = bufs[b]
            isn, idn, ewn, rgn, _, gsemn = bufs[1 - b]
            r = 2 * k + b
            wait_rows(gsem, rg)
            if b == 0:
                @pl.when(k >= 1)
                def _():
                    wait_rows(ssem, rgn)
                load_chunk(r + 1, isn, idn, ewn)
                pltpu.async_copy(table.at[isn], rgn, gsemn)
            else:
                wait_rows(ssem, rgn)

                @pl.when(k < FULL_ROUNDS // 2 - 1)
                def _():
                    load_chunk(r + 1, isn, idn, ewn)
                    pltpu.async_copy(table.at[isn], rgn, gsemn)
            scale(idv, ewv, rg, ds)
            pltpu.async_copy(rg, acc_sh.at[ds], ssem, add=True)
        return carry

    lax.fori_loop(0, FULL_ROUNDS // 2, body, 0)
    wait_rows(ssem, rg1)

    @pl.when(wid < EXTRA)
    def _():
        load_chunk(FULL_ROUNDS, is0, id0, ew0)
        pltpu.async_copy(table.at[is0], rg0, gsem0).wait()
        scale(id0, ew0, rg0, ds0)
        pltpu.sync_copy(rg0, acc_sh.at[ds0], add=True)

    plsc.subcore_barrier()

    pltpu.sync_copy(acc_sh.at[pl.ds(s * N_PER_T, N_PER_T)],
                    out.at[c, pl.ds(s * N_PER_T, N_PER_T)])
    pltpu.sync_copy(cnt_v, out_cnt.at[c, s, 0])


@functools.partial(
    pl.kernel,
    out_type=jax.ShapeDtypeStruct((2 * B, D), jnp.float32),
    mesh=_mesh,
    scratch_types=[
        pltpu.VMEM((GCH,), jnp.int32),
        pltpu.VMEM((GCH, D), jnp.float32),
        pltpu.SemaphoreType.DMA,
    ],
)
def _sc_gather(table, idx, out, idx_v, rows_v, sem):
    c = lax.axis_index("c")
    s = lax.axis_index("s")
    wid = c * NS + s

    @pl.when(wid < 2 * B // GCH)
    def _():
        base = wid * GCH
        pltpu.sync_copy(idx.at[pl.ds(base, GCH)], idx_v)
        pltpu.async_copy(table.at[idx_v], rows_v, sem).wait()
        pltpu.sync_copy(rows_v, out.at[pl.ds(base, GCH)])


def _proj_body(x_ref, w_ref, b_ref, out_ref):
    out_ref[...] = lax.dot_general(
        x_ref[...], w_ref[...], (((1,), (1,)), ((), ())),
        preferred_element_type=jnp.float32) + b_ref[...]


def _ew_body(ea_ref, q_ref, p0_ref, p1_ref, b0_ref, b1_ref, out_ref):
    wq = jnp.concatenate([p0_ref[...] * q_ref[...], p1_ref[...] * q_ref[...]],
                         axis=0)
    z = lax.dot_general(ea_ref[...], wq, (((1,), (1,)), ((), ())),
                        preferred_element_type=jnp.float32)
    b = jnp.concatenate([b0_ref[...], b1_ref[...]], axis=1)
    out_ref[...] = jax.nn.sigmoid(z + b)


def _gru_body(a0_ref, a1_ref, cnt_ref, h_ref, mw_ref, mb_ref, wih_ref,
              bih_ref, whh_ref, bhh_ref, out_ref):
    a = a0_ref[0] + a1_ref[0]
    mb32 = jnp.broadcast_to(mb_ref[...], (NW, D))
    bias = lax.dot_general(cnt_ref[...], mb32, (((0,), (0,)), ((), ())),
                           preferred_element_type=jnp.float32)
    aggr = lax.dot_general(a, mw_ref[...], (((1,), (1,)), ((), ())),
                           preferred_element_type=jnp.float32) + bias
    h = h_ref[...]
    gi = lax.dot_general(aggr, wih_ref[...], (((1,), (1,)), ((), ())),
                         preferred_element_type=jnp.float32) + bih_ref[...]
    gh = lax.dot_general(h, whh_ref[...], (((1,), (1,)), ((), ())),
                         preferred_element_type=jnp.float32) + bhh_ref[...]
    r = jax.nn.sigmoid(gi[:, :D] + gh[:, :D])
    z = jax.nn.sigmoid(gi[:, D:2 * D] + gh[:, D:2 * D])
    n = jnp.tanh(gi[:, 2 * D:] + r * gh[:, 2 * D:])
    out_ref[...] = (1.0 - z) * n + z * h


def _score_body(hsd_ref, lab_ref, q_ref, w1_ref, b1_ref, w2_ref, b2_ref,
                out_ref):
    hs = hsd_ref[:B, :]
    hd = hsd_ref[B:, :]
    qb = jnp.broadcast_to(q_ref[...], (B, D))
    feat = jnp.concatenate([hs, hd, hs * hd, qb], axis=1)
    hid = jax.nn.relu(
        lax.dot_general(feat, w1_ref[...], (((1,), (1,)), ((), ())),
                        preferred_element_type=jnp.float32) + b1_ref[...])
    logits = lax.dot_general(w2_ref[...], hid, (((1,), (1,)), ((), ())),
                             preferred_element_type=jnp.float32) + b2_ref[...]
    y = lab_ref[...].astype(jnp.float32)
    l = (jnp.maximum(logits, 0.0) - logits * y
         + jnp.log1p(jnp.exp(-jnp.abs(logits))))
    out_ref[...] = jnp.mean(l)[None, None]


def _full(shape):
    return pl.BlockSpec(shape, lambda i: tuple(0 for _ in shape))


_proj_call = pl.pallas_call(
    _proj_body,
    grid=(NPAD // NBLK,),
    in_specs=[
        pl.BlockSpec((NBLK, D), lambda i: (i, 0)),
        _full((D, D)),
        _full((1, D)),
    ],
    out_specs=pl.BlockSpec((NBLK, D), lambda i: (i, 0)),
    out_shape=jax.ShapeDtypeStruct((NPAD, D), jnp.float32),
)

_ew_call = pl.pallas_call(
    _ew_body,
    grid=(E // EBLK,),
    in_specs=[
        pl.BlockSpec((EBLK, D), lambda i: (i, 0)),
        _full((1, D)),
        _full((1, D)),
        _full((1, D)),
        _full((1, 1)),
        _full((1, 1)),
    ],
    out_specs=pl.BlockSpec((EBLK, 2), lambda i: (i, 0)),
    out_shape=jax.ShapeDtypeStruct((E, 2), jnp.float32),
)

_gru_call = pl.pallas_call(
    _gru_body,
    grid=(NPAD // NBLK,),
    in_specs=[
        pl.BlockSpec((1, NBLK, D), lambda i: (0, i, 0)),
        pl.BlockSpec((1, NBLK, D), lambda i: (1, i, 0)),
        pl.BlockSpec((NW, NBLK), lambda i: (0, i)),
        pl.BlockSpec((NBLK, D), lambda i: (i, 0)),
        _full((D, D)),
        _full((1, D)),
        _full((3 * D, D)),
        _full((1, 3 * D)),
        _full((3 * D, D)),
        _full((1, 3 * D)),
    ],
    out_specs=pl.BlockSpec((NBLK, D), lambda i: (i, 0)),
    out_shape=jax.ShapeDtypeStruct((NPAD, D), jnp.float32),
)

_score_call = pl.pallas_call(
    _score_body,
    grid=(1,),
    in_specs=[
        pl.BlockSpec((2 * B, D), lambda i: (0, 0)),
        pl.BlockSpec((1, B), lambda i: (0, 0)),
        _full((1, D)),
        _full((D, 4 * D)),
        _full((1, D)),
        _full((1, D)),
        _full((1, 1)),
    ],
    out_specs=pl.BlockSpec((1, 1), lambda i: (0, 0)),
    out_shape=jax.ShapeDtypeStruct((1, 1), jnp.float32),
)


def kernel(x, edge_index, edge_attr, q_emb, src, dst, labels, proj_W, proj_b,
           l0_msg_W, l0_msg_b, l0_phi_W, l0_phi_b, l0_gru_Wih, l0_gru_bih,
           l0_gru_Whh, l0_gru_bhh, l1_msg_W, l1_msg_b, l1_phi_W, l1_phi_b,
           l1_gru_Wih, l1_gru_bih, l1_gru_Whh, l1_gru_bhh, edge_W1, edge_b1,
           edge_W2, edge_b2):
    s_idx = edge_index[0]
    d_idx = edge_index[1]
    xp = jnp.pad(x, ((0, NPAD - N), (0, 0)))

    h0 = _proj_call(xp, proj_W, proj_b.reshape(1, D))
    ew = _ew_call(edge_attr, q_emb, l0_phi_W, l1_phi_W,
                  l0_phi_b.reshape(1, 1), l1_phi_b.reshape(1, 1))

    acc0, cnt0 = _sc_scatter(h0, s_idx, d_idx, ew[:, 0])
    h1 = _gru_call(acc0, acc0, cnt0.reshape(NW, NPAD), h0,
                   l0_msg_W, l0_msg_b.reshape(1, D),
                   l0_gru_Wih, l0_gru_bih.reshape(1, 3 * D),
                   l0_gru_Whh, l0_gru_bhh.reshape(1, 3 * D))

    acc1, cnt1 = _sc_scatter(h1, s_idx, d_idx, ew[:, 1])
    h2 = _gru_call(acc1, acc1, cnt1.reshape(NW, NPAD), h1,
                   l1_msg_W, l1_msg_b.reshape(1, D),
                   l1_gru_Wih, l1_gru_bih.reshape(1, 3 * D),
                   l1_gru_Whh, l1_gru_bhh.reshape(1, 3 * D))

    hsd = _sc_gather(h2, jnp.concatenate([src, dst]))
    loss = _score_call(hsd, labels.reshape(1, B), q_emb, edge_W1,
                       edge_b1.reshape(1, D), edge_W2, edge_b2.reshape(1, 1))
    return loss[0, 0]


# speedup vs baseline: 1.7441x; 1.3282x over previous
import functools

import jax
import jax.numpy as jnp
from jax import lax
from jax.experimental import pallas as pl
from jax.experimental.pallas import tpu as pltpu
from jax.experimental.pallas import tpu_sc as plsc

N = 10000
E = 320000
D = 128
B = 1024
NPAD = 10240
NC = 2
NS = 16
NW = NC * NS
CH = 128
NCHUNKS = E // CH
FULL_ROUNDS = NCHUNKS // NW
EXTRA = NCHUNKS - FULL_ROUNDS * NW
N_PER_T = NPAD // NS
GCH = 128
NBLK = 1024
EBLK = 4000

_mesh = plsc.VectorSubcoreMesh(core_axis_name="c", subcore_axis_name="s")


@functools.partial(
    pl.kernel,
    out_type=(
        jax.ShapeDtypeStruct((NC, NPAD, D), jnp.float32),
        jax.ShapeDtypeStruct((NC, NS, 1, NPAD), jnp.float32),
    ),
    mesh=_mesh,
    scratch_types=[
        pltpu.VMEM((3, CH), jnp.int32),
        pltpu.VMEM((CH, D), jnp.float32),
        pltpu.VMEM((CH,), jnp.int32),
        pltpu.VMEM((3, CH), jnp.int32),
        pltpu.VMEM((CH, D), jnp.float32),
        pltpu.VMEM((CH,), jnp.int32),
        pltpu.VMEM((NPAD,), jnp.float32),
        pltpu.VMEM_SHARED((NPAD, D), jnp.float32),
        pltpu.SemaphoreType.DMA,
        pltpu.SemaphoreType.DMA,
        pltpu.SemaphoreType.DMA,
        pltpu.SemaphoreType.DMA,
        pltpu.SemaphoreType.DMA,
    ],
    compiler_params=pltpu.CompilerParams(needs_layout_passes=False),
)
def _sc_scatter(table, payload, out, out_cnt,
                pay0, rg0, ds0, pay1, rg1, ds1,
                cnt_v, acc_sh, gsem0, gsem1, psem0, psem1, ssem):
    c = lax.axis_index("c")
    s = lax.axis_index("s")
    wid = c * NS + s
    zeros16 = jnp.zeros((16,), jnp.float32)
    bufs = ((pay0, rg0, ds0, gsem0, psem0),
            (pay1, rg1, ds1, gsem1, psem1))

    def zcnt(i, carry):
        cnt_v[pl.ds(i * 16, 16)] = zeros16
        return carry

    lax.fori_loop(0, NPAD // 16, zcnt, 0)

    def zrow(i, carry):
        for j in range(D // 16):
            rg0[i, pl.ds(j * 16, 16)] = zeros16
        return carry

    lax.fori_loop(0, CH, zrow, 0)
    for k in range(N_PER_T // CH):
        pltpu.sync_copy(rg0, acc_sh.at[pl.ds(s * N_PER_T + k * CH, CH)])
    plsc.subcore_barrier()

    def scale(pay, rg, ds):
        def grp(g, carry):
            ew16 = plsc.bitcast(pay[2, pl.ds(g * 16, 16)], jnp.float32)
            d16 = pay[1, pl.ds(g * 16, 16)]
            ds[pl.ds(g * 16, 16)] = d16
            plsc.addupdate_scatter(cnt_v, [d16], ew16)
            for j16 in range(16):
                i = g * 16 + j16
                w = lax.gather(
                    ew16, jnp.full((16, 1), j16, jnp.int32),
                    lax.GatherDimensionNumbers(
                        offset_dims=(), collapsed_slice_dims=(0,),
                        start_index_map=(0,)),
                    (1,), mode=lax.GatherScatterMode.PROMISE_IN_BOUNDS)
                for j in range(D // 16):
                    rg[i, pl.ds(j * 16, 16)] = rg[i, pl.ds(j * 16, 16)] * w
            return carry

        lax.fori_loop(0, CH // 16, grp, 0)

    def wait_rows(sem, buf):
        pltpu.make_async_copy(table.at[pl.ds(0, CH)], buf, sem).wait()

    def wait_pay(sem, buf):
        pltpu.make_async_copy(payload.at[0], buf, sem).wait()

    def chunk_slot(r):
        return r * NW + wid

    pltpu.sync_copy(payload.at[chunk_slot(0)], pay0)
    pltpu.async_copy(table.at[pay0.at[0]], rg0, gsem0)
    pltpu.async_copy(payload.at[chunk_slot(1)], pay1, psem1)

    def body(k, carry):
        for b in range(2):
            pay, rg, ds, gsem, psem = bufs[b]
            payn, rgn, _, gsemn, psemn = bufs[1 - b]
            r = 2 * k + b
            wait_rows(gsem, rg)
            if b == 0:
                @pl.when(k >= 1)
                def _():
                    wait_rows(ssem, rgn)
                wait_pay(psemn, payn)
                pltpu.async_copy(table.at[payn.at[0]], rgn, gsemn)
            else:
                wait_rows(ssem, rgn)

                @pl.when(k < FULL_ROUNDS // 2 - 1)
                def _():
                    wait_pay(psemn, payn)
                    pltpu.async_copy(table.at[payn.at[0]], rgn, gsemn)
            scale(pay, rg, ds)
            @pl.when(k < FULL_ROUNDS // 2 - 1)
            def _():
                pltpu.async_copy(payload.at[chunk_slot(r + 2)], pay, psem)
            pltpu.async_copy(rg, acc_sh.at[ds], ssem, add=True)
        return carry

    lax.fori_loop(0, FULL_ROUNDS // 2, body, 0)
    wait_rows(ssem, rg1)

    @pl.when(wid < EXTRA)
    def _():
        pltpu.sync_copy(payload.at[FULL_ROUNDS * NW + wid], pay0)
        pltpu.async_copy(table.at[pay0.at[0]], rg0, gsem0).wait()
        scale(pay0, rg0, ds0)
        pltpu.sync_copy(rg0, acc_sh.at[ds0], add=True)

    plsc.subcore_barrier()

    pltpu.sync_copy(acc_sh.at[pl.ds(s * N_PER_T, N_PER_T)],
                    out.at[c, pl.ds(s * N_PER_T, N_PER_T)])
    pltpu.sync_copy(cnt_v, out_cnt.at[c, s, 0])


@functools.partial(
    pl.kernel,
    out_type=jax.ShapeDtypeStruct((2 * B, D), jnp.float32),
    mesh=_mesh,
    scratch_types=[
        pltpu.VMEM((GCH,), jnp.int32),
        pltpu.VMEM((GCH, D), jnp.float32),
        pltpu.SemaphoreType.DMA,
    ],
)
def _sc_gather(table, idx, out, idx_v, rows_v, sem):
    c = lax.axis_index("c")
    s = lax.axis_index("s")
    wid = c * NS + s

    @pl.when(wid < 2 * B // GCH)
    def _():
        base = wid * GCH
        pltpu.sync_copy(idx.at[pl.ds(base, GCH)], idx_v)
        pltpu.async_copy(table.at[idx_v], rows_v, sem).wait()
        pltpu.sync_copy(rows_v, out.at[pl.ds(base, GCH)])


def _proj_body(x_ref, w_ref, b_ref, out_ref):
    out_ref[...] = lax.dot_general(
        x_ref[...], w_ref[...], (((1,), (1,)), ((), ())),
        preferred_element_type=jnp.float32) + b_ref[...]


def _ew_body(ea_ref, q_ref, p0_ref, p1_ref, b0_ref, b1_ref, out_ref):
    wq = jnp.concatenate([p0_ref[...] * q_ref[...], p1_ref[...] * q_ref[...]],
                         axis=0)
    z = lax.dot_general(ea_ref[...], wq, (((1,), (1,)), ((), ())),
                        preferred_element_type=jnp.float32)
    b = jnp.concatenate([b0_ref[...], b1_ref[...]], axis=1)
    out_ref[...] = jax.nn.sigmoid(z + b)


def _gru_body(a0_ref, a1_ref, cnt_ref, h_ref, mw_ref, mb_ref, wih_ref,
              bih_ref, whh_ref, bhh_ref, out_ref):
    a = a0_ref[0] + a1_ref[0]
    mb32 = jnp.broadcast_to(mb_ref[...], (NW, D))
    bias = lax.dot_general(cnt_ref[...], mb32, (((0,), (0,)), ((), ())),
                           preferred_element_type=jnp.float32)
    aggr = lax.dot_general(a, mw_ref[...], (((1,), (1,)), ((), ())),
                           preferred_element_type=jnp.float32) + bias
    h = h_ref[...]
    gi = lax.dot_general(aggr, wih_ref[...], (((1,), (1,)), ((), ())),
                         preferred_element_type=jnp.float32) + bih_ref[...]
    gh = lax.dot_general(h, whh_ref[...], (((1,), (1,)), ((), ())),
                         preferred_element_type=jnp.float32) + bhh_ref[...]
    r = jax.nn.sigmoid(gi[:, :D] + gh[:, :D])
    z = jax.nn.sigmoid(gi[:, D:2 * D] + gh[:, D:2 * D])
    n = jnp.tanh(gi[:, 2 * D:] + r * gh[:, 2 * D:])
    out_ref[...] = (1.0 - z) * n + z * h


def _score_body(hsd_ref, lab_ref, q_ref, w1_ref, b1_ref, w2_ref, b2_ref,
                out_ref):
    hs = hsd_ref[:B, :]
    hd = hsd_ref[B:, :]
    qb = jnp.broadcast_to(q_ref[...], (B, D))
    feat = jnp.concatenate([hs, hd, hs * hd, qb], axis=1)
    hid = jax.nn.relu(
        lax.dot_general(feat, w1_ref[...], (((1,), (1,)), ((), ())),
                        preferred_element_type=jnp.float32) + b1_ref[...])
    logits = lax.dot_general(w2_ref[...], hid, (((1,), (1,)), ((), ())),
                             preferred_element_type=jnp.float32) + b2_ref[...]
    y = lab_ref[...].astype(jnp.float32)
    l = (jnp.maximum(logits, 0.0) - logits * y
         + jnp.log1p(jnp.exp(-jnp.abs(logits))))
    out_ref[...] = jnp.mean(l)[None, None]


def _full(shape):
    return pl.BlockSpec(shape, lambda i: tuple(0 for _ in shape))


_proj_call = pl.pallas_call(
    _proj_body,
    grid=(NPAD // NBLK,),
    in_specs=[
        pl.BlockSpec((NBLK, D), lambda i: (i, 0)),
        _full((D, D)),
        _full((1, D)),
    ],
    out_specs=pl.BlockSpec((NBLK, D), lambda i: (i, 0)),
    out_shape=jax.ShapeDtypeStruct((NPAD, D), jnp.float32),
)

_ew_call = pl.pallas_call(
    _ew_body,
    grid=(E // EBLK,),
    in_specs=[
        pl.BlockSpec((EBLK, D), lambda i: (i, 0)),
        _full((1, D)),
        _full((1, D)),
        _full((1, D)),
        _full((1, 1)),
        _full((1, 1)),
    ],
    out_specs=pl.BlockSpec((EBLK, 2), lambda i: (i, 0)),
    out_shape=jax.ShapeDtypeStruct((E, 2), jnp.float32),
)

_gru_call = pl.pallas_call(
    _gru_body,
    grid=(NPAD // NBLK,),
    in_specs=[
        pl.BlockSpec((1, NBLK, D), lambda i: (0, i, 0)),
        pl.BlockSpec((1, NBLK, D), lambda i: (1, i, 0)),
        pl.BlockSpec((NW, NBLK), lambda i: (0, i)),
        pl.BlockSpec((NBLK, D), lambda i: (i, 0)),
        _full((D, D)),
        _full((1, D)),
        _full((3 * D, D)),
        _full((1, 3 * D)),
        _full((3 * D, D)),
        _full((1, 3 * D)),
    ],
    out_specs=pl.BlockSpec((NBLK, D), lambda i: (i, 0)),
    out_shape=jax.ShapeDtypeStruct((NPAD, D), jnp.float32),
)

_score_call = pl.pallas_call(
    _score_body,
    grid=(1,),
    in_specs=[
        pl.BlockSpec((2 * B, D), lambda i: (0, 0)),
        pl.BlockSpec((1, B), lambda i: (0, 0)),
        _full((1, D)),
        _full((D, 4 * D)),
        _full((1, D)),
        _full((1, D)),
        _full((1, 1)),
    ],
    out_specs=pl.BlockSpec((1, 1), lambda i: (0, 0)),
    out_shape=jax.ShapeDtypeStruct((1, 1), jnp.float32),
)


def kernel(x, edge_index, edge_attr, q_emb, src, dst, labels, proj_W, proj_b,
           l0_msg_W, l0_msg_b, l0_phi_W, l0_phi_b, l0_gru_Wih, l0_gru_bih,
           l0_gru_Whh, l0_gru_bhh, l1_msg_W, l1_msg_b, l1_phi_W, l1_phi_b,
           l1_gru_Wih, l1_gru_bih, l1_gru_Whh, l1_gru_bhh, edge_W1, edge_b1,
           edge_W2, edge_b2):
    def payload(ew_col):
        p = jnp.stack([edge_index[0], edge_index[1],
                       lax.bitcast_convert_type(ew_col, jnp.int32)])
        return p.reshape(3, NCHUNKS, CH).transpose(1, 0, 2)

    xp = jnp.pad(x, ((0, NPAD - N), (0, 0)))

    h0 = _proj_call(xp, proj_W, proj_b.reshape(1, D))
    ew = _ew_call(edge_attr, q_emb, l0_phi_W, l1_phi_W,
                  l0_phi_b.reshape(1, 1), l1_phi_b.reshape(1, 1))

    acc0, cnt0 = _sc_scatter(h0, payload(ew[:, 0]))
    h1 = _gru_call(acc0, acc0, cnt0.reshape(NW, NPAD), h0,
                   l0_msg_W, l0_msg_b.reshape(1, D),
                   l0_gru_Wih, l0_gru_bih.reshape(1, 3 * D),
                   l0_gru_Whh, l0_gru_bhh.reshape(1, 3 * D))

    acc1, cnt1 = _sc_scatter(h1, payload(ew[:, 1]))
    h2 = _gru_call(acc1, acc1, cnt1.reshape(NW, NPAD), h1,
                   l1_msg_W, l1_msg_b.reshape(1, D),
                   l1_gru_Wih, l1_gru_bih.reshape(1, 3 * D),
                   l1_gru_Whh, l1_gru_bhh.reshape(1, 3 * D))

    hsd = _sc_gather(h2, jnp.concatenate([src, dst]))
    loss = _score_call(hsd, labels.reshape(1, B), q_emb, edge_W1,
                       edge_b1.reshape(1, D), edge_W2, edge_b2.reshape(1, 1))
    return loss[0, 0]
